# baseline (device time: 467589 ns/iter reference)
import jax
import jax.numpy as jnp
from jax import lax
from jax.experimental import pallas as pl
from jax.experimental.pallas import tpu as pltpu

N_DEV = 8
N_EXP = 64
N_LOC = N_EXP // N_DEV
CAP = 64
D_IN = 512
D_OUT = 1024
N_TOK = 2048

_MESH = pl.DeviceIdType.MESH


def _body(send_ref, x_ref, sw_ref, ew_ref, res_ref, sh_ref,
          recv_ref, y_ref, dsend, drecv, rsend, rrecv, lsem):
    me = lax.axis_index("i")

    barrier = pltpu.get_barrier_semaphore()
    for d in range(N_DEV):
        @pl.when(d != me)
        def _():
            pl.semaphore_signal(barrier, inc=1, device_id=(d,),
                                device_id_type=_MESH)
    pl.semaphore_wait(barrier, N_DEV - 1)

    self_in = pltpu.make_async_copy(
        send_ref.at[pl.ds(me * N_LOC, N_LOC)], recv_ref.at[me], lsem)
    self_in.start()
    for d in range(N_DEV):
        @pl.when(d != me)
        def _():
            pltpu.make_async_remote_copy(
                src_ref=send_ref.at[pl.ds(d * N_LOC, N_LOC)],
                dst_ref=recv_ref.at[me],
                send_sem=dsend.at[d],
                recv_sem=drecv.at[me],
                device_id=(d,), device_id_type=_MESH,
            ).start()

    for b in range(4):
        rows = pl.ds(b * (N_TOK // 4), N_TOK // 4)
        sh_ref[rows, :] = jnp.dot(x_ref[rows, :], sw_ref[:, :],
                                  preferred_element_type=jnp.float32)

    for s in range(N_DEV):
        @pl.when(s != me)
        def _():
            pltpu.make_async_remote_copy(
                src_ref=send_ref.at[pl.ds(s * N_LOC, N_LOC)],
                dst_ref=recv_ref.at[s],
                send_sem=dsend.at[s],
                recv_sem=drecv.at[s],
                device_id=(s,), device_id_type=_MESH,
            ).wait_recv()
    self_in.wait()

    for el in range(N_LOC):
        xe = recv_ref[:, el, :, :].reshape(N_DEV * CAP, D_IN)
        ye = jnp.dot(xe, ew_ref[el], preferred_element_type=jnp.float32)
        y_ref[:, el, :, :] = ye.reshape(N_DEV, CAP, D_OUT)

    self_out = pltpu.make_async_copy(y_ref.at[me], res_ref.at[me], lsem)
    self_out.start()
    for d in range(N_DEV):
        @pl.when(d != me)
        def _():
            pltpu.make_async_remote_copy(
                src_ref=y_ref.at[d],
                dst_ref=res_ref.at[me],
                send_sem=rsend.at[d],
                recv_sem=rrecv.at[me],
                device_id=(d,), device_id_type=_MESH,
            ).start()
    for s in range(N_DEV):
        @pl.when(s != me)
        def _():
            pltpu.make_async_remote_copy(
                src_ref=y_ref.at[s],
                dst_ref=res_ref.at[s],
                send_sem=rsend.at[s],
                recv_sem=rrecv.at[s],
                device_id=(s,), device_id_type=_MESH,
            ).wait_recv()
    self_out.wait()

    for d in range(N_DEV):
        @pl.when(d != me)
        def _():
            pltpu.make_async_remote_copy(
                src_ref=send_ref.at[pl.ds(d * N_LOC, N_LOC)],
                dst_ref=recv_ref.at[me], send_sem=dsend.at[d],
                recv_sem=drecv.at[me], device_id=(d,), device_id_type=_MESH,
            ).wait_send()
            pltpu.make_async_remote_copy(
                src_ref=y_ref.at[d],
                dst_ref=res_ref.at[me], send_sem=rsend.at[d],
                recv_sem=rrecv.at[me], device_id=(d,), device_id_type=_MESH,
            ).wait_send()


def kernel(x, router_W, route_idx, expert_W, shared_W):
    n, d_in = x.shape
    e = route_idx[:, 0]
    scores = x @ router_W
    p = jax.nn.softmax(scores, axis=-1)
    p_sel = jnp.take_along_axis(p, route_idx, axis=1)[:, 0]
    xs = x * p_sel[:, None]

    onehot = (e[:, None] == jnp.arange(N_EXP, dtype=e.dtype)[None, :])
    rank = jnp.take_along_axis(jnp.cumsum(onehot.astype(jnp.int32), axis=0),
                               route_idx, axis=1)[:, 0] - 1
    slot = e * CAP + jnp.minimum(rank, CAP - 1)
    send_buf = (jnp.zeros((N_EXP * CAP, d_in), jnp.float32)
                .at[slot].set(xs)
                .reshape(N_EXP, CAP, d_in))

    res, shared = pl.pallas_call(
        _body,
        out_shape=[
            jax.ShapeDtypeStruct((N_DEV, N_LOC, CAP, D_OUT), jnp.float32),
            jax.ShapeDtypeStruct((n, D_OUT), jnp.float32),
        ],
        in_specs=[
            pl.BlockSpec(memory_space=pl.ANY),
            pl.BlockSpec(memory_space=pltpu.VMEM),
            pl.BlockSpec(memory_space=pltpu.VMEM),
            pl.BlockSpec(memory_space=pltpu.VMEM),
        ],
        out_specs=[
            pl.BlockSpec(memory_space=pl.ANY),
            pl.BlockSpec(memory_space=pltpu.VMEM),
        ],
        scratch_shapes=[
            pltpu.VMEM((N_DEV, N_LOC, CAP, D_IN), jnp.float32),
            pltpu.VMEM((N_DEV, N_LOC, CAP, D_OUT), jnp.float32),
            pltpu.SemaphoreType.DMA((N_DEV,)),
            pltpu.SemaphoreType.DMA((N_DEV,)),
            pltpu.SemaphoreType.DMA((N_DEV,)),
            pltpu.SemaphoreType.DMA((N_DEV,)),
            pltpu.SemaphoreType.DMA,
        ],
        compiler_params=pltpu.CompilerParams(collective_id=0),
    )(send_buf, x, shared_W, expert_W)

    return shared + res.reshape(N_EXP * CAP, D_OUT)[slot]


# device time: 160656 ns/iter; 2.9105x vs baseline; 2.9105x over previous
import jax
import jax.numpy as jnp
from jax import lax
from jax.experimental import pallas as pl
from jax.experimental.pallas import tpu as pltpu

N_DEV = 8
N_EXP = 64
N_LOC = N_EXP // N_DEV
CAP = 64
BLK = N_LOC * CAP
D_IN = 512
D_OUT = 1024
N_TOK = 2048
TOK_BLK = 512

_MESH = pl.DeviceIdType.MESH


def _onehot_block(slot, lo, n_rows, n_cols, row0):
    s = slot[pl.ds(row0, n_rows)]
    cols = lax.broadcasted_iota(jnp.int32, (n_rows, n_cols), 1) + lo
    return (s == cols).astype(jnp.bfloat16)


def _body(x_ref, rw_ref, idx_ref, slot_ref, sw_ref, ew_hbm, out_ref,
          xs_ref, recv_ref, y_ref, res_ref, ewbuf, dbuf,
          dsend, drecv, rsend, rrecv, ewsems):
    me = lax.axis_index("i")

    ew0 = pltpu.make_async_copy(ew_hbm.at[0], ewbuf.at[0], ewsems.at[0])
    ew0.start()

    barrier = pltpu.get_barrier_semaphore()
    for d in range(N_DEV):
        @pl.when(d != me)
        def _():
            pl.semaphore_signal(barrier, inc=1, device_id=(d,),
                                device_id_type=_MESH)

    scores = jnp.dot(x_ref[...], rw_ref[...],
                     preferred_element_type=jnp.float32)
    m = jnp.max(scores, axis=1, keepdims=True)
    ex = jnp.exp(scores - m)
    sel = idx_ref[...] == lax.broadcasted_iota(jnp.int32, (N_TOK, N_EXP), 1)
    p_sel = (jnp.sum(jnp.where(sel, ex, 0.0), axis=1, keepdims=True)
             / jnp.sum(ex, axis=1, keepdims=True))
    xs_ref[...] = (x_ref[...] * p_sel).astype(jnp.bfloat16)

    pl.semaphore_wait(barrier, N_DEV - 1)

    xs = xs_ref[...]
    for k in range(N_DEV):
        d = (me + k) % N_DEV
        p_d = _onehot_block(slot_ref, d * BLK, N_TOK, BLK, 0)
        send_d = lax.dot_general(
            p_d, xs, (((0,), (0,)), ((), ())),
            preferred_element_type=jnp.float32)
        send_d = send_d.astype(jnp.bfloat16)
        if k == 0:
            recv_ref[pl.ds(me, 1)] = send_d.reshape(1, N_LOC, CAP, D_IN)
        else:
            dbuf[k] = send_d.reshape(N_LOC, CAP, D_IN)
            pltpu.make_async_remote_copy(
                src_ref=dbuf.at[k],
                dst_ref=recv_ref.at[me],
                send_sem=dsend.at[k],
                recv_sem=drecv.at[k],
                device_id=(d,), device_id_type=_MESH,
            ).start()

    for b in range(N_TOK // TOK_BLK):
        rows = pl.ds(b * TOK_BLK, TOK_BLK)
        out_ref[rows, :] = jnp.dot(x_ref[rows, :], sw_ref[...],
                                   preferred_element_type=jnp.float32)

    for k in range(1, N_DEV):
        s = (me - k) % N_DEV
        pltpu.make_async_remote_copy(
            src_ref=dbuf.at[k],
            dst_ref=recv_ref.at[s],
            send_sem=dsend.at[k],
            recv_sem=drecv.at[k],
            device_id=(s,), device_id_type=_MESH,
        ).wait_recv()

    for el in range(N_LOC):
        pltpu.make_async_copy(
            ew_hbm.at[el], ewbuf.at[el % 2], ewsems.at[el % 2]).wait()
        if el + 1 < N_LOC:
            pltpu.make_async_copy(
                ew_hbm.at[el + 1], ewbuf.at[(el + 1) % 2],
                ewsems.at[(el + 1) % 2]).start()
        xe = recv_ref[:, el, :, :].reshape(N_DEV * CAP, D_IN)
        ye = jnp.dot(xe.astype(jnp.float32), ewbuf[el % 2],
                     preferred_element_type=jnp.float32)
        y_ref[:, el, :, :] = ye.astype(jnp.bfloat16).reshape(N_DEV, CAP, D_OUT)

    res_ref[pl.ds(me, 1)] = y_ref[pl.ds(me, 1)]
    for k in range(1, N_DEV):
        d = (me + k) % N_DEV
        pltpu.make_async_remote_copy(
            src_ref=y_ref.at[d],
            dst_ref=res_ref.at[me],
            send_sem=rsend.at[k],
            recv_sem=rrecv.at[k],
            device_id=(d,), device_id_type=_MESH,
        ).start()

    for k in range(N_DEV):
        d2 = (me - k) % N_DEV
        if k > 0:
            pltpu.make_async_remote_copy(
                src_ref=y_ref.at[d2],
                dst_ref=res_ref.at[d2],
                send_sem=rsend.at[k],
                recv_sem=rrecv.at[k],
                device_id=(d2,), device_id_type=_MESH,
            ).wait_recv()
        rd = res_ref[pl.ds(d2, 1)].reshape(BLK, D_OUT)
        for b in range(N_TOK // TOK_BLK):
            p_b = _onehot_block(slot_ref, d2 * BLK, TOK_BLK, BLK, b * TOK_BLK)
            rows = pl.ds(b * TOK_BLK, TOK_BLK)
            out_ref[rows, :] += jnp.dot(p_b, rd,
                                        preferred_element_type=jnp.float32)

    for k in range(1, N_DEV):
        pltpu.make_async_remote_copy(
            src_ref=dbuf.at[k], dst_ref=recv_ref.at[me],
            send_sem=dsend.at[k], recv_sem=drecv.at[k],
            device_id=((me + k) % N_DEV,), device_id_type=_MESH,
        ).wait_send()
        pltpu.make_async_remote_copy(
            src_ref=y_ref.at[(me + k) % N_DEV], dst_ref=res_ref.at[me],
            send_sem=rsend.at[k], recv_sem=rrecv.at[k],
            device_id=((me + k) % N_DEV,), device_id_type=_MESH,
        ).wait_send()


def kernel(x, router_W, route_idx, expert_W, shared_W):
    n, d_in = x.shape
    e = route_idx[:, 0].astype(jnp.int32)

    oh = route_idx == jnp.arange(N_EXP, dtype=jnp.int32)[None, :]
    csum = jnp.cumsum(oh.astype(jnp.int32), axis=0)
    rank = jnp.sum(jnp.where(oh, csum, 0), axis=1) - 1
    slot = e * CAP + jnp.minimum(rank, CAP - 1)
    slot = slot[:, None].astype(jnp.int32)

    return pl.pallas_call(
        _body,
        out_shape=jax.ShapeDtypeStruct((n, D_OUT), jnp.float32),
        in_specs=[
            pl.BlockSpec(memory_space=pltpu.VMEM),
            pl.BlockSpec(memory_space=pltpu.VMEM),
            pl.BlockSpec(memory_space=pltpu.VMEM),
            pl.BlockSpec(memory_space=pltpu.VMEM),
            pl.BlockSpec(memory_space=pltpu.VMEM),
            pl.BlockSpec(memory_space=pl.ANY),
        ],
        out_specs=pl.BlockSpec(memory_space=pltpu.VMEM),
        scratch_shapes=[
            pltpu.VMEM((N_TOK, D_IN), jnp.bfloat16),
            pltpu.VMEM((N_DEV, N_LOC, CAP, D_IN), jnp.bfloat16),
            pltpu.VMEM((N_DEV, N_LOC, CAP, D_OUT), jnp.bfloat16),
            pltpu.VMEM((N_DEV, N_LOC, CAP, D_OUT), jnp.bfloat16),
            pltpu.VMEM((2, D_IN, D_OUT), jnp.float32),
            pltpu.VMEM((N_DEV, N_LOC, CAP, D_IN), jnp.bfloat16),
            pltpu.SemaphoreType.DMA((N_DEV,)),
            pltpu.SemaphoreType.DMA((N_DEV,)),
            pltpu.SemaphoreType.DMA((N_DEV,)),
            pltpu.SemaphoreType.DMA((N_DEV,)),
            pltpu.SemaphoreType.DMA((2,)),
        ],
        compiler_params=pltpu.CompilerParams(
            collective_id=0,
            vmem_limit_bytes=100 * 1024 * 1024,
        ),
    )(x, router_W, route_idx.astype(jnp.int32), slot, shared_W, expert_W)
